# baseline (speedup 1.0000x reference)
import functools

import jax
import jax.numpy as jnp
from jax import lax
from jax.experimental import pallas as pl
from jax.experimental.pallas import tpu as pltpu
from jax.experimental.pallas import tpu_sc as plsc

_H = 8
_LN = 16
_NC = 2
_NS = 16
_QT = 32
_NB = 4


def _mm_body(x_ref, w_ref, b_ref, o_ref):
    o_ref[...] = (
        jnp.dot(x_ref[...], w_ref[...], preferred_element_type=jnp.float32,
                precision=lax.Precision.DEFAULT)
        + b_ref[...]
    )


def _matmul_bias(x, wt, b, bm=1024):
    n, c = x.shape
    co = wt.shape[1]
    return pl.pallas_call(
        _mm_body,
        grid=(n // bm,),
        in_specs=[
            pl.BlockSpec((bm, c), lambda i: (i, 0)),
            pl.BlockSpec((c, co), lambda i: (0, 0)),
            pl.BlockSpec((1, co), lambda i: (0, 0)),
        ],
        out_specs=pl.BlockSpec((bm, co), lambda i: (i, 0)),
        out_shape=jax.ShapeDtypeStruct((n, co), jnp.float32),
    )(x, wt, b.reshape(1, co))


def _kv_body(k_ref, v_ref, wk_ref, wv_ref, bk_ref, bv_ref, o_ref):
    c = k_ref.shape[1]
    hc = c // 2
    kp = (
        jnp.dot(k_ref[...], wk_ref[...], preferred_element_type=jnp.float32,
                precision=lax.Precision.DEFAULT)
        + bk_ref[...]
    )
    vp = (
        jnp.dot(v_ref[...], wv_ref[...], preferred_element_type=jnp.float32,
                precision=lax.Precision.DEFAULT)
        + bv_ref[...]
    )
    o_ref[0, :, :hc] = kp[:, :hc]
    o_ref[0, :, hc:] = vp[:, :hc]
    o_ref[1, :, :hc] = kp[:, hc:]
    o_ref[1, :, hc:] = vp[:, hc:]


def _kv_proj(key, value, wk_t, wv_t, bk, bv, bm=1024):
    m, c = key.shape
    return pl.pallas_call(
        _kv_body,
        grid=(m // bm,),
        in_specs=[
            pl.BlockSpec((bm, c), lambda i: (i, 0)),
            pl.BlockSpec((bm, c), lambda i: (i, 0)),
            pl.BlockSpec((c, c), lambda i: (0, 0)),
            pl.BlockSpec((c, c), lambda i: (0, 0)),
            pl.BlockSpec((1, c), lambda i: (0, 0)),
            pl.BlockSpec((1, c), lambda i: (0, 0)),
        ],
        out_specs=pl.BlockSpec((2, bm, c), lambda i: (0, i, 0)),
        out_shape=jax.ShapeDtypeStruct((2, m, c), jnp.float32),
    )(key, value, wk_t, wv_t, bk.reshape(1, c), bv.reshape(1, c))


def _out_body(oh_ref, wh_ref, w0_ref, w1_ref, b_ref, o_ref, ws_ref):
    o_ref[...] = (
        jnp.dot(oh_ref[0], w0_ref[...], preferred_element_type=jnp.float32,
                precision=lax.Precision.DEFAULT)
        + jnp.dot(oh_ref[1], w1_ref[...], preferred_element_type=jnp.float32,
                  precision=lax.Precision.DEFAULT)
        + b_ref[...]
    )
    ws_ref[...] = wh_ref[0] + wh_ref[1]


def _out_proj(out_half, wsum_half, w_t, b, bm=1024):
    _, n, hc = out_half.shape
    c = w_t.shape[1]
    return pl.pallas_call(
        _out_body,
        grid=(n // bm,),
        in_specs=[
            pl.BlockSpec((2, bm, hc), lambda i: (0, i, 0)),
            pl.BlockSpec((2, bm, _LN), lambda i: (0, i, 0)),
            pl.BlockSpec((hc, c), lambda i: (0, 0)),
            pl.BlockSpec((hc, c), lambda i: (0, 0)),
            pl.BlockSpec((1, c), lambda i: (0, 0)),
        ],
        out_specs=[
            pl.BlockSpec((bm, c), lambda i: (i, 0)),
            pl.BlockSpec((bm, _LN), lambda i: (i, 0)),
        ],
        out_shape=[
            jax.ShapeDtypeStruct((n, c), jnp.float32),
            jax.ShapeDtypeStruct((n, _LN), jnp.float32),
        ],
    )(out_half, wsum_half, w_t[:hc], w_t[hc:], b.reshape(1, c))


def _sc_attention(qp, tt, lidx, maskf, nbatch):
    n, c = qp.shape
    hc = c // 2
    dh = c // _H
    nsub = dh // 16
    hh = _H // 2
    nqb = n // nbatch
    qpt = nqb // _NS

    mesh = plsc.VectorSubcoreMesh(core_axis_name="c", subcore_axis_name="s")

    @functools.partial(
        pl.kernel,
        out_type=(
            jax.ShapeDtypeStruct((2, n, hc), jnp.float32),
            jax.ShapeDtypeStruct((2, n, _LN), jnp.float32),
        ),
        mesh=mesh,
        scratch_types=[
            pltpu.VMEM_SHARED((nqb, c), jnp.float32),
            pltpu.VMEM((qpt, _LN), jnp.int32),
            pltpu.VMEM((qpt, _LN), jnp.float32),
            pltpu.VMEM((_QT, c), jnp.float32),
            pltpu.VMEM((_LN, c), jnp.float32),
            pltpu.VMEM((_LN, c), jnp.float32),
            pltpu.VMEM((_LN, c), jnp.float32),
            pltpu.VMEM((_LN, c), jnp.float32),
            pltpu.VMEM((_QT, hc), jnp.float32),
            pltpu.VMEM((_QT, _LN), jnp.float32),
            pltpu.SemaphoreType.DMA,
            pltpu.SemaphoreType.DMA,
            pltpu.SemaphoreType.DMA,
            pltpu.SemaphoreType.DMA,
        ],
        compiler_params=pltpu.CompilerParams(
            use_tc_tiling_on_sc=False, needs_layout_passes=False),
    )
    def attn(qp_hbm, tt_hbm, lidx_hbm, maskf_hbm, out_hbm, wsum_hbm,
             spm, idx_v, mask_v, q_v, kv_0, kv_1, kv_2, kv_3, o_v, ws_v,
             sem_0, sem_1, sem_2, sem_3):
        core = lax.axis_index("c")
        tile = lax.axis_index("s")
        chan0 = core * hc

        kv_bufs = (kv_0, kv_1, kv_2, kv_3)
        sems = (sem_0, sem_1, sem_2, sem_3)

        lane_ids = [jnp.full((16,), l, jnp.int32) for l in range(_LN)]
        lane15 = lane_ids[15]
        eqmask = [lax.iota(jnp.int32, 16) == l for l in range(_LN)]
        inv_h = jnp.float32(1.0 / _H)

        def batch(b, carry):
            @pl.when(tile == 0)
            def _():
                pltpu.sync_copy(tt_hbm.at[core, pl.ds(b * nqb, nqb)], spm)

            qbase = b * nqb + tile * qpt
            pltpu.sync_copy(lidx_hbm.at[pl.ds(qbase, qpt)], idx_v)
            pltpu.sync_copy(maskf_hbm.at[pl.ds(qbase, qpt)], mask_v)
            plsc.subcore_barrier()

            for p in range(_NB - 1):
                pltpu.async_copy(spm.at[idx_v.at[p]], kv_bufs[p], sems[p])

            def process(i, sub):
                kv_buf, sem = kv_bufs[sub], sems[sub]
                kv_nbuf = kv_bufs[(sub + _NB - 1) % _NB]
                sem_n = sems[(sub + _NB - 1) % _NB]
                qi = lax.rem(i, _QT)
                ipn = i + _NB - 1

                @pl.when(qi == 0)
                def _():
                    pltpu.sync_copy(qp_hbm.at[pl.ds(qbase + i, _QT)], q_v)

                @pl.when(ipn < qpt)
                def _():
                    pltpu.async_copy(spm.at[idx_v.at[ipn]], kv_nbuf, sem_n)

                pltpu.make_async_copy(spm.at[idx_v.at[i]], kv_buf, sem).wait()

                pad = mask_v[i, :] > 0.5

                def one_head(h):
                    hoff = h * dh
                    qvs = [q_v[qi, pl.ds(chan0 + hoff + 16 * j, 16)]
                           for j in range(nsub)]
                    acc = jnp.zeros((16,), jnp.float32)
                    for l in range(_LN):
                        prods = [kv_buf[l, pl.ds(hoff + 16 * j, 16)] * qvs[j]
                                 for j in range(nsub)]
                        tot = (prods[0] + prods[1]) + (prods[2] + prods[3])
                        red = plsc.cumsum(tot)[lane15]
                        acc = jnp.where(eqmask[l], red, acc)
                    sv = jnp.where(pad, jnp.float32(-1000.0), acc)
                    e = jnp.exp(sv - jnp.max(sv))
                    w = e / jnp.sum(e)
                    va = [jnp.zeros((16,), jnp.float32) for _ in range(nsub)]
                    vb = [jnp.zeros((16,), jnp.float32) for _ in range(nsub)]
                    for l in range(_LN):
                        wl = w[lane_ids[l]]
                        dst = va if l % 2 == 0 else vb
                        for j in range(nsub):
                            dst[j] = dst[j] + wl * kv_buf[l, pl.ds(hc + hoff + 16 * j, 16)]
                    for j in range(nsub):
                        o_v[qi, pl.ds(hoff + 16 * j, 16)] = va[j] + vb[j]
                    return w

                ws = [one_head(h) for h in range(hh)]
                wacc = (ws[0] + ws[1]) + (ws[2] + ws[3])
                ws_v[qi, :] = wacc * inv_h

                @pl.when(qi == _QT - 1)
                def _():
                    rs = qbase + i - (_QT - 1)
                    pltpu.sync_copy(o_v, out_hbm.at[core, pl.ds(rs, _QT)])
                    pltpu.sync_copy(ws_v, wsum_hbm.at[core, pl.ds(rs, _QT)])

            def quad(g, carry2):
                for sub in range(_NB):
                    process(_NB * g + sub, sub)
                return carry2

            lax.fori_loop(0, qpt // _NB, quad, 0)
            plsc.subcore_barrier()
            return carry

        lax.fori_loop(0, nbatch, batch, 0)

    return attn(qp, tt, lidx, maskf)


def kernel(query, key, value, index_pair, query_batch_cnt, key_batch_cnt,
           index_pair_batch, in_proj_weight, in_proj_bias, out_proj_weight,
           out_proj_bias):
    n, c = query.shape
    nbatch = query_batch_cnt.shape[0]
    dh = c // _H
    scaling = float(dh) ** -0.5

    wq_t = in_proj_weight[:c].T * scaling
    wk_t = in_proj_weight[c:2 * c].T
    wv_t = in_proj_weight[2 * c:].T
    bq = in_proj_bias[:c] * scaling
    bk = in_proj_bias[c:2 * c]
    bv = in_proj_bias[2 * c:]

    mask = index_pair < 0
    lidx = jnp.where(mask, 0, index_pair)
    maskf = mask.astype(jnp.float32)

    qp = _matmul_bias(query, wq_t, bq)
    tt = _kv_proj(key, value, wk_t, wv_t, bk, bv)

    out_half, wsum_half = _sc_attention(qp, tt, lidx, maskf, nbatch)

    out, wsum = _out_proj(out_half, wsum_half, out_proj_weight.T, out_proj_bias)
    return out, wsum


# speedup vs baseline: 1.8042x; 1.8042x over previous
import functools

import jax
import jax.numpy as jnp
from jax import lax
from jax.experimental import pallas as pl
from jax.experimental.pallas import tpu as pltpu
from jax.experimental.pallas import tpu_sc as plsc

_H = 8
_LN = 16
_NC = 2
_NS = 16
_QT = 32
_NB = 2


def _mm_body(x_ref, w_ref, b_ref, o_ref):
    o_ref[...] = (
        jnp.dot(x_ref[...], w_ref[...], preferred_element_type=jnp.float32,
                precision=lax.Precision.DEFAULT)
        + b_ref[...]
    )


def _matmul_bias(x, wt, b, bm=1024):
    n, c = x.shape
    co = wt.shape[1]
    return pl.pallas_call(
        _mm_body,
        grid=(n // bm,),
        in_specs=[
            pl.BlockSpec((bm, c), lambda i: (i, 0)),
            pl.BlockSpec((c, co), lambda i: (0, 0)),
            pl.BlockSpec((1, co), lambda i: (0, 0)),
        ],
        out_specs=pl.BlockSpec((bm, co), lambda i: (i, 0)),
        out_shape=jax.ShapeDtypeStruct((n, co), jnp.float32),
    )(x, wt, b.reshape(1, co))


def _kv_body(k_ref, v_ref, wk_ref, wv_ref, bk_ref, bv_ref, o_ref):
    c = k_ref.shape[1]
    hc = c // 2
    kp = (
        jnp.dot(k_ref[...], wk_ref[...], preferred_element_type=jnp.float32,
                precision=lax.Precision.DEFAULT)
        + bk_ref[...]
    )
    vp = (
        jnp.dot(v_ref[...], wv_ref[...], preferred_element_type=jnp.float32,
                precision=lax.Precision.DEFAULT)
        + bv_ref[...]
    )
    o_ref[0, :, :hc] = kp[:, :hc]
    o_ref[0, :, hc:] = vp[:, :hc]
    o_ref[1, :, :hc] = kp[:, hc:]
    o_ref[1, :, hc:] = vp[:, hc:]


def _kv_proj(key, value, wk_t, wv_t, bk, bv, bm=1024):
    m, c = key.shape
    return pl.pallas_call(
        _kv_body,
        grid=(m // bm,),
        in_specs=[
            pl.BlockSpec((bm, c), lambda i: (i, 0)),
            pl.BlockSpec((bm, c), lambda i: (i, 0)),
            pl.BlockSpec((c, c), lambda i: (0, 0)),
            pl.BlockSpec((c, c), lambda i: (0, 0)),
            pl.BlockSpec((1, c), lambda i: (0, 0)),
            pl.BlockSpec((1, c), lambda i: (0, 0)),
        ],
        out_specs=pl.BlockSpec((2, bm, c), lambda i: (0, i, 0)),
        out_shape=jax.ShapeDtypeStruct((2, m, c), jnp.float32),
    )(key, value, wk_t, wv_t, bk.reshape(1, c), bv.reshape(1, c))


def _out_body(oh_ref, wh_ref, w0_ref, w1_ref, b_ref, o_ref, ws_ref):
    o_ref[...] = (
        jnp.dot(oh_ref[0], w0_ref[...], preferred_element_type=jnp.float32,
                precision=lax.Precision.DEFAULT)
        + jnp.dot(oh_ref[1], w1_ref[...], preferred_element_type=jnp.float32,
                  precision=lax.Precision.DEFAULT)
        + b_ref[...]
    )
    ws_ref[...] = wh_ref[0] + wh_ref[1]


def _out_proj(out_half, wsum_half, w_t, b, bm=1024):
    _, n, hc = out_half.shape
    c = w_t.shape[1]
    return pl.pallas_call(
        _out_body,
        grid=(n // bm,),
        in_specs=[
            pl.BlockSpec((2, bm, hc), lambda i: (0, i, 0)),
            pl.BlockSpec((2, bm, _LN), lambda i: (0, i, 0)),
            pl.BlockSpec((hc, c), lambda i: (0, 0)),
            pl.BlockSpec((hc, c), lambda i: (0, 0)),
            pl.BlockSpec((1, c), lambda i: (0, 0)),
        ],
        out_specs=[
            pl.BlockSpec((bm, c), lambda i: (i, 0)),
            pl.BlockSpec((bm, _LN), lambda i: (i, 0)),
        ],
        out_shape=[
            jax.ShapeDtypeStruct((n, c), jnp.float32),
            jax.ShapeDtypeStruct((n, _LN), jnp.float32),
        ],
    )(out_half, wsum_half, w_t[:hc], w_t[hc:], b.reshape(1, c))


def _sc_attention(qp, tt, lidx, maskf, nbatch):
    n, c = qp.shape
    hc = c // 2
    dh = c // _H
    nsub = dh // 16
    hh = _H // 2
    nqb = n // nbatch
    qpt = nqb // _NS

    mesh = plsc.VectorSubcoreMesh(core_axis_name="c", subcore_axis_name="s")

    @functools.partial(
        pl.kernel,
        out_type=(
            jax.ShapeDtypeStruct((2, n, hc), jnp.float32),
            jax.ShapeDtypeStruct((2, n, _LN), jnp.float32),
        ),
        mesh=mesh,
        scratch_types=[
            pltpu.VMEM_SHARED((nqb, c), jnp.float32),
            pltpu.VMEM((qpt, _LN), jnp.int32),
            pltpu.VMEM((qpt, _LN), jnp.float32),
            pltpu.VMEM((_QT, c), jnp.float32),
            pltpu.VMEM((_LN, c), jnp.float32),
            pltpu.VMEM((_LN, c), jnp.float32),
            pltpu.VMEM((_LN, c), jnp.float32),
            pltpu.VMEM((_LN, c), jnp.float32),
            pltpu.VMEM((_QT, hc), jnp.float32),
            pltpu.VMEM((_QT, _LN), jnp.float32),
            pltpu.SemaphoreType.DMA,
            pltpu.SemaphoreType.DMA,
            pltpu.SemaphoreType.DMA,
            pltpu.SemaphoreType.DMA,
        ],
        compiler_params=pltpu.CompilerParams(
            use_tc_tiling_on_sc=False, needs_layout_passes=False),
    )
    def attn(qp_hbm, tt_hbm, lidx_hbm, maskf_hbm, out_hbm, wsum_hbm,
             spm, idx_v, mask_v, q_v, kv_0, kv_1, kv_2, kv_3, o_v, ws_v,
             sem_0, sem_1, sem_2, sem_3):
        core = lax.axis_index("c")
        tile = lax.axis_index("s")
        chan0 = core * hc

        kv_bufs = (kv_0, kv_1, kv_2, kv_3)[:_NB]
        sems = (sem_0, sem_1, sem_2, sem_3)[:_NB]

        lane_ids = [jnp.full((16,), l, jnp.int32) for l in range(_LN)]
        lane15 = lane_ids[15]
        eqmask = [lax.iota(jnp.int32, 16) == l for l in range(_LN)]
        inv_h = jnp.float32(1.0 / _H)

        def batch(b, carry):
            @pl.when(tile == 0)
            def _():
                pltpu.sync_copy(tt_hbm.at[core, pl.ds(b * nqb, nqb)], spm)

            qbase = b * nqb + tile * qpt
            pltpu.sync_copy(lidx_hbm.at[pl.ds(qbase, qpt)], idx_v)
            pltpu.sync_copy(maskf_hbm.at[pl.ds(qbase, qpt)], mask_v)
            plsc.subcore_barrier()

            for p in range(_NB - 1):
                pltpu.async_copy(spm.at[idx_v.at[p]], kv_bufs[p], sems[p])

            def process(i, sub):
                kv_buf, sem = kv_bufs[sub], sems[sub]
                kv_nbuf = kv_bufs[(sub + _NB - 1) % _NB]
                sem_n = sems[(sub + _NB - 1) % _NB]
                qi = lax.rem(i, _QT)
                ipn = i + _NB - 1

                @pl.when(qi == 0)
                def _():
                    pltpu.sync_copy(qp_hbm.at[pl.ds(qbase + i, _QT)], q_v)

                @pl.when(ipn < qpt)
                def _():
                    pltpu.async_copy(spm.at[idx_v.at[ipn]], kv_nbuf, sem_n)

                pltpu.make_async_copy(spm.at[idx_v.at[i]], kv_buf, sem).wait()

                pad = mask_v[i, :] > 0.5

                def one_head(h):
                    hoff = h * dh
                    qvs = [q_v[qi, pl.ds(chan0 + hoff + 16 * j, 16)]
                           for j in range(nsub)]
                    acc = jnp.zeros((16,), jnp.float32)
                    for l in range(_LN):
                        prods = [kv_buf[l, pl.ds(hoff + 16 * j, 16)] * qvs[j]
                                 for j in range(nsub)]
                        tot = (prods[0] + prods[1]) + (prods[2] + prods[3])
                        red = plsc.cumsum(tot)[lane15]
                        acc = jnp.where(eqmask[l], red, acc)
                    sv = jnp.where(pad, jnp.float32(-1000.0), acc)
                    e = jnp.exp(sv - jnp.max(sv))
                    w = e / jnp.sum(e)
                    va = [jnp.zeros((16,), jnp.float32) for _ in range(nsub)]
                    vb = [jnp.zeros((16,), jnp.float32) for _ in range(nsub)]
                    for l in range(_LN):
                        wl = w[lane_ids[l]]
                        dst = va if l % 2 == 0 else vb
                        for j in range(nsub):
                            dst[j] = dst[j] + wl * kv_buf[l, pl.ds(hc + hoff + 16 * j, 16)]
                    for j in range(nsub):
                        o_v[qi, pl.ds(hoff + 16 * j, 16)] = va[j] + vb[j]
                    return w

                def head2(g, wacc):
                    w0 = one_head(2 * g)
                    w1 = one_head(2 * g + 1)
                    return wacc + w0 + w1

                wacc = lax.fori_loop(0, hh // 2, head2,
                                     jnp.zeros((16,), jnp.float32))
                ws_v[qi, :] = wacc * inv_h

                @pl.when(qi == _QT - 1)
                def _():
                    rs = qbase + i - (_QT - 1)
                    pltpu.sync_copy(o_v, out_hbm.at[core, pl.ds(rs, _QT)])
                    pltpu.sync_copy(ws_v, wsum_hbm.at[core, pl.ds(rs, _QT)])

            def quad(g, carry2):
                for sub in range(_NB):
                    process(_NB * g + sub, sub)
                return carry2

            lax.fori_loop(0, qpt // _NB, quad, 0)
            plsc.subcore_barrier()
            return carry

        lax.fori_loop(0, nbatch, batch, 0)

    return attn(qp, tt, lidx, maskf)


def kernel(query, key, value, index_pair, query_batch_cnt, key_batch_cnt,
           index_pair_batch, in_proj_weight, in_proj_bias, out_proj_weight,
           out_proj_bias):
    n, c = query.shape
    nbatch = query_batch_cnt.shape[0]
    dh = c // _H
    scaling = float(dh) ** -0.5

    wq_t = in_proj_weight[:c].T * scaling
    wk_t = in_proj_weight[c:2 * c].T
    wv_t = in_proj_weight[2 * c:].T
    bq = in_proj_bias[:c] * scaling
    bk = in_proj_bias[c:2 * c]
    bv = in_proj_bias[2 * c:]

    mask = index_pair < 0
    lidx = jnp.where(mask, 0, index_pair)
    maskf = mask.astype(jnp.float32)

    qp = _matmul_bias(query, wq_t, bq)
    tt = _kv_proj(key, value, wk_t, wv_t, bk, bv)

    out_half, wsum_half = _sc_attention(qp, tt, lidx, maskf, nbatch)

    out, wsum = _out_proj(out_half, wsum_half, out_proj_weight.T, out_proj_bias)
    return out, wsum
